# baseline (speedup 1.0000x reference)
import functools

import jax
import jax.numpy as jnp
from jax import lax
from jax.experimental import pallas as pl
from jax.experimental.pallas import tpu as pltpu
from jax.experimental.pallas import tpu_sc as plsc

_B = 4096
_T = 200
_H = 64
_NW = 32
_RPW = _B // _NW
_NBUF = 4
_LANES = 16
_CCH = _H // _LANES
_TC0, _TC1 = 128, _T - 128


def _pool_body(entity_hbm, table_hbm, out_hbm, eidx, bufs, out_v, sems):
    c = lax.axis_index("c")
    s = lax.axis_index("s")
    wid = s * 2 + c
    base = wid * _RPW

    pltpu.sync_copy(entity_hbm.at[pl.ds(base, _RPW)], eidx)

    inv_t = jnp.float32(1.0 / _T)

    def issue(row, buf, sem):
        r = jnp.minimum(row, _RPW - 1)
        d0 = pltpu.async_copy(
            table_hbm.at[eidx.at[r, pl.ds(0, _TC0)]],
            buf.at[pl.ds(0, _TC0)], sem)
        d1 = pltpu.async_copy(
            table_hbm.at[eidx.at[r, pl.ds(_TC0, _TC1)]],
            buf.at[pl.ds(_TC0, _TC1)], sem)
        return d0, d1

    def accum(row, buf):
        def tstep(t, accs):
            return tuple(
                accs[cc] + buf[t, pl.ds(cc * _LANES, _LANES)]
                for cc in range(_CCH))
        accs = lax.fori_loop(
            0, _T, tstep,
            tuple(jnp.zeros((_LANES,), jnp.float32) for _ in range(_CCH)),
            unroll=2)
        for cc in range(_CCH):
            out_v[row, pl.ds(cc * _LANES, _LANES)] = accs[cc] * inv_t

    def group(i, carry):
        r0 = i * _NBUF
        descs = []
        for k in range(_NBUF):
            descs.append(issue(r0 + k, bufs[k], sems[k]))
        for k in range(_NBUF):
            d0, d1 = descs[k]
            d0.wait()
            d1.wait()
            accum(r0 + k, bufs[k])
        return carry

    lax.fori_loop(0, _RPW // _NBUF, group, 0)
    pltpu.sync_copy(out_v, out_hbm.at[pl.ds(base, _RPW)])


@functools.partial(jax.jit, static_argnums=())
def _pool(entity, emb):
    mesh = plsc.VectorSubcoreMesh(core_axis_name="c", subcore_axis_name="s")
    f = pl.kernel(
        _pool_body,
        out_type=jax.ShapeDtypeStruct((_B, _H), jnp.float32),
        mesh=mesh,
        scratch_types=[
            pltpu.VMEM((_RPW, _T), jnp.int32),
            [pltpu.VMEM((_T, _H), jnp.float32) for _ in range(_NBUF)],
            pltpu.VMEM((_RPW, _H), jnp.float32),
            [pltpu.SemaphoreType.DMA for _ in range(_NBUF)],
        ],
        compiler_params=pltpu.CompilerParams(use_tc_tiling_on_sc=False),
    )
    return f(entity, emb)


_VB = 4096
_V = 1000000
_NBLK = (_V + _VB - 1) // _VB
_VROWS = _NBLK * _VB


def _tr_body(in_ref, out_ref):
    x = in_ref[...]
    xt = jnp.transpose(x, (1, 0))
    out_ref[...] = jnp.concatenate(
        [xt[: _VB // 2], xt[_VB // 2:]], axis=1)


def _detile(embT):
    return pl.pallas_call(
        _tr_body,
        grid=(_NBLK,),
        in_specs=[pl.BlockSpec((_H, _VB), lambda i: (0, i))],
        out_specs=pl.BlockSpec((_VB // 2, 128), lambda i: (i, 0)),
        out_shape=jax.ShapeDtypeStruct((_VROWS // 2, 128), jnp.float32),
        compiler_params=pltpu.CompilerParams(
            dimension_semantics=("parallel",)),
    )(embT)


def _remap(entity):
    r = entity & (_VB - 1)
    return entity + jnp.where(r < _VB // 2, r, r - (_VB - 1))


def _mlp_body(x_ref, w0, b0, w1, b1, w2, b2, w3, b3, w4, b4, out_ref):
    x = x_ref[...]
    x = jnp.maximum(jnp.dot(x, w0[...], preferred_element_type=jnp.float32)
                    + b0[...], 0.0)
    x = jnp.maximum(jnp.dot(x, w1[...], preferred_element_type=jnp.float32)
                    + b1[...], 0.0)
    x = jnp.maximum(jnp.dot(x, w2[...], preferred_element_type=jnp.float32)
                    + b2[...], 0.0)
    x = jnp.maximum(jnp.dot(x, w3[...], preferred_element_type=jnp.float32)
                    + b3[...], 0.0)
    out_ref[...] = (jnp.dot(x, w4[...], preferred_element_type=jnp.float32)
                    + b4[...])


def _mlp(x, W0, b0, W1, b1, W2, b2, W3, b3, W4, b4):
    bb = 512
    n_out = W4.shape[1]
    grid = (_B // bb,)

    def wspec(w):
        return pl.BlockSpec(w.shape, lambda i: (0,) * w.ndim)

    return pl.pallas_call(
        _mlp_body,
        grid=grid,
        in_specs=[pl.BlockSpec((bb, _H), lambda i: (i, 0))]
        + [wspec(w) for w in (W0, b0, W1, b1, W2, b2, W3, b3, W4, b4)],
        out_specs=pl.BlockSpec((bb, n_out), lambda i: (i, 0)),
        out_shape=jax.ShapeDtypeStruct((_B, n_out), jnp.float32),
    )(x, W0, b0, W1, b1, W2, b2, W3, b3, W4, b4)


def kernel(entity, emb, W0, b0, W1, b1, W2, b2, W3, b3, W4, b4):
    emb_lin = _detile(emb.T).reshape(_VROWS, _H)
    pooled = _pool(_remap(entity), emb_lin)
    return _mlp(pooled, W0, b0, W1, b1, W2, b2, W3, b3, W4, b4)


# speedup vs baseline: 1.1458x; 1.1458x over previous
import functools

import jax
import jax.numpy as jnp
from jax import lax
from jax.experimental import pallas as pl
from jax.experimental.pallas import tpu as pltpu
from jax.experimental.pallas import tpu_sc as plsc

_B = 4096
_T = 200
_H = 64
_NW = 32
_RPW = _B // _NW
_NBUF = 4
_LANES = 16
_CCH = _H // _LANES
_TC0, _TC1 = 128, _T - 128


def _pool_body(entity_hbm, table_hbm, out_hbm, eidx, bufs, out_v, sems):
    c = lax.axis_index("c")
    s = lax.axis_index("s")
    wid = s * 2 + c
    base = wid * _RPW

    pltpu.sync_copy(entity_hbm.at[pl.ds(base, _RPW)], eidx)

    inv_t = jnp.float32(1.0 / _T)

    def issue(row, buf, sem):
        r = jnp.minimum(row, _RPW - 1)
        d0 = pltpu.async_copy(
            table_hbm.at[eidx.at[r, pl.ds(0, _TC0)]],
            buf.at[pl.ds(0, _TC0)], sem)
        d1 = pltpu.async_copy(
            table_hbm.at[eidx.at[r, pl.ds(_TC0, _TC1)]],
            buf.at[pl.ds(_TC0, _TC1)], sem)
        return d0, d1

    def accum(row, buf):
        def tstep(t, accs):
            return tuple(
                accs[cc] + buf[t, pl.ds(cc * _LANES, _LANES)]
                for cc in range(_CCH))
        accs = lax.fori_loop(
            0, _T, tstep,
            tuple(jnp.zeros((_LANES,), jnp.float32) for _ in range(_CCH)),
            unroll=2)
        for cc in range(_CCH):
            out_v[row, pl.ds(cc * _LANES, _LANES)] = accs[cc] * inv_t

    def group(i, carry):
        r0 = i * _NBUF
        descs = []
        for k in range(_NBUF):
            descs.append(issue(r0 + k, bufs[k], sems[k]))
        for k in range(_NBUF):
            d0, d1 = descs[k]
            d0.wait()
            d1.wait()
            accum(r0 + k, bufs[k])
        return carry

    lax.fori_loop(0, _RPW // _NBUF, group, 0)
    pltpu.sync_copy(out_v, out_hbm.at[pl.ds(base, _RPW)])


@functools.partial(jax.jit, static_argnums=())
def _pool(entity, emb):
    mesh = plsc.VectorSubcoreMesh(core_axis_name="c", subcore_axis_name="s")
    f = pl.kernel(
        _pool_body,
        out_type=jax.ShapeDtypeStruct((_B, _H), jnp.float32),
        mesh=mesh,
        scratch_types=[
            pltpu.VMEM((_RPW, _T), jnp.int32),
            [pltpu.VMEM((_T, _H), jnp.float32) for _ in range(_NBUF)],
            pltpu.VMEM((_RPW, _H), jnp.float32),
            [pltpu.SemaphoreType.DMA for _ in range(_NBUF)],
        ],
        compiler_params=pltpu.CompilerParams(use_tc_tiling_on_sc=False),
    )
    return f(entity, emb)


_VB = 8192
_V = 1000000
_NBLK = (_V + _VB - 1) // _VB
_VROWS = _NBLK * _VB


def _tr_body(in_ref, out_ref):
    x = in_ref[...]
    xt = jnp.transpose(x, (1, 0))
    out_ref[...] = jnp.concatenate(
        [xt[: _VB // 2], xt[_VB // 2:]], axis=1)


def _detile(embT):
    return pl.pallas_call(
        _tr_body,
        grid=(_NBLK,),
        in_specs=[pl.BlockSpec((_H, _VB), lambda i: (0, i))],
        out_specs=pl.BlockSpec((_VB // 2, 128), lambda i: (i, 0)),
        out_shape=jax.ShapeDtypeStruct((_VROWS // 2, 128), jnp.float32),
        compiler_params=pltpu.CompilerParams(
            dimension_semantics=("parallel",)),
    )(embT)


def _remap(entity):
    r = entity & (_VB - 1)
    return entity + jnp.where(r < _VB // 2, r, r - (_VB - 1))


def _mlp_body(x_ref, w0, b0, w1, b1, w2, b2, w3, b3, w4, b4, out_ref):
    x = x_ref[...]
    x = jnp.maximum(jnp.dot(x, w0[...], preferred_element_type=jnp.float32)
                    + b0[...], 0.0)
    x = jnp.maximum(jnp.dot(x, w1[...], preferred_element_type=jnp.float32)
                    + b1[...], 0.0)
    x = jnp.maximum(jnp.dot(x, w2[...], preferred_element_type=jnp.float32)
                    + b2[...], 0.0)
    x = jnp.maximum(jnp.dot(x, w3[...], preferred_element_type=jnp.float32)
                    + b3[...], 0.0)
    out_ref[...] = (jnp.dot(x, w4[...], preferred_element_type=jnp.float32)
                    + b4[...])


def _mlp(x, W0, b0, W1, b1, W2, b2, W3, b3, W4, b4):
    bb = 512
    n_out = W4.shape[1]
    grid = (_B // bb,)

    def wspec(w):
        return pl.BlockSpec(w.shape, lambda i: (0,) * w.ndim)

    return pl.pallas_call(
        _mlp_body,
        grid=grid,
        in_specs=[pl.BlockSpec((bb, _H), lambda i: (i, 0))]
        + [wspec(w) for w in (W0, b0, W1, b1, W2, b2, W3, b3, W4, b4)],
        out_specs=pl.BlockSpec((bb, n_out), lambda i: (i, 0)),
        out_shape=jax.ShapeDtypeStruct((_B, n_out), jnp.float32),
    )(x, W0, b0, W1, b1, W2, b2, W3, b3, W4, b4)


def kernel(entity, emb, W0, b0, W1, b1, W2, b2, W3, b3, W4, b4):
    emb_lin = _detile(emb.T).reshape(_VROWS, _H)
    pooled = _pool(_remap(entity), emb_lin)
    return _mlp(pooled, W0, b0, W1, b1, W2, b2, W3, b3, W4, b4)


# speedup vs baseline: 1.2251x; 1.0692x over previous
import functools

import jax
import jax.numpy as jnp
from jax import lax
from jax.experimental import pallas as pl
from jax.experimental.pallas import tpu as pltpu
from jax.experimental.pallas import tpu_sc as plsc

_B = 4096
_T = 200
_H = 64
_NW = 32
_RPW = _B // _NW
_NBUF = 4
_LANES = 16
_CCH = _H // _LANES
_TC0, _TC1 = 128, _T - 128


def _pool_body(entity_hbm, table_hbm, out_hbm, eidx, bufs, out_v, sems):
    c = lax.axis_index("c")
    s = lax.axis_index("s")
    wid = s * 2 + c
    base = wid * _RPW

    pltpu.sync_copy(entity_hbm.at[pl.ds(base, _RPW)], eidx)

    inv_t = jnp.float32(1.0 / _T)

    def issue(row, buf, sem):
        r = jnp.minimum(row, _RPW - 1)
        d0 = pltpu.async_copy(
            table_hbm.at[eidx.at[r, pl.ds(0, _TC0)]],
            buf.at[pl.ds(0, _TC0)], sem)
        d1 = pltpu.async_copy(
            table_hbm.at[eidx.at[r, pl.ds(_TC0, _TC1)]],
            buf.at[pl.ds(_TC0, _TC1)], sem)
        return d0, d1

    def accum(row, buf):
        def tstep(t, accs):
            return tuple(
                accs[cc] + buf[t, pl.ds(cc * _LANES, _LANES)]
                for cc in range(_CCH))
        accs = lax.fori_loop(
            0, _T, tstep,
            tuple(jnp.zeros((_LANES,), jnp.float32) for _ in range(_CCH)),
            unroll=2)
        for cc in range(_CCH):
            out_v[row, pl.ds(cc * _LANES, _LANES)] = accs[cc] * inv_t

    def group(i, carry):
        r0 = i * _NBUF
        descs = []
        for k in range(_NBUF):
            descs.append(issue(r0 + k, bufs[k], sems[k]))
        for k in range(_NBUF):
            d0, d1 = descs[k]
            d0.wait()
            d1.wait()
            accum(r0 + k, bufs[k])
        return carry

    lax.fori_loop(0, _RPW // _NBUF, group, 0)
    pltpu.sync_copy(out_v, out_hbm.at[pl.ds(base, _RPW)])


@functools.partial(jax.jit, static_argnums=())
def _pool(entity, emb):
    mesh = plsc.VectorSubcoreMesh(core_axis_name="c", subcore_axis_name="s")
    f = pl.kernel(
        _pool_body,
        out_type=jax.ShapeDtypeStruct((_B, _H), jnp.float32),
        mesh=mesh,
        scratch_types=[
            pltpu.VMEM((_RPW, _T), jnp.int32),
            [pltpu.VMEM((_T, _H), jnp.float32) for _ in range(_NBUF)],
            pltpu.VMEM((_RPW, _H), jnp.float32),
            [pltpu.SemaphoreType.DMA for _ in range(_NBUF)],
        ],
        compiler_params=pltpu.CompilerParams(use_tc_tiling_on_sc=False),
    )
    return f(entity, emb)


_VB = 16384
_V = 1000000
_NBLK = (_V + _VB - 1) // _VB
_VROWS = _NBLK * _VB


def _tr_body(in_ref, out_ref):
    x = in_ref[...]
    xt = jnp.transpose(x, (1, 0))
    out_ref[...] = jnp.concatenate(
        [xt[: _VB // 2], xt[_VB // 2:]], axis=1)


def _detile(embT):
    return pl.pallas_call(
        _tr_body,
        grid=(_NBLK,),
        in_specs=[pl.BlockSpec((_H, _VB), lambda i: (0, i))],
        out_specs=pl.BlockSpec((_VB // 2, 128), lambda i: (i, 0)),
        out_shape=jax.ShapeDtypeStruct((_VROWS // 2, 128), jnp.float32),
        compiler_params=pltpu.CompilerParams(
            dimension_semantics=("parallel",)),
    )(embT)


def _remap(entity):
    r = entity & (_VB - 1)
    return entity + jnp.where(r < _VB // 2, r, r - (_VB - 1))


def _mlp_body(x_ref, w0, b0, w1, b1, w2, b2, w3, b3, w4, b4, out_ref):
    x = x_ref[...]
    x = jnp.maximum(jnp.dot(x, w0[...], preferred_element_type=jnp.float32)
                    + b0[...], 0.0)
    x = jnp.maximum(jnp.dot(x, w1[...], preferred_element_type=jnp.float32)
                    + b1[...], 0.0)
    x = jnp.maximum(jnp.dot(x, w2[...], preferred_element_type=jnp.float32)
                    + b2[...], 0.0)
    x = jnp.maximum(jnp.dot(x, w3[...], preferred_element_type=jnp.float32)
                    + b3[...], 0.0)
    out_ref[...] = (jnp.dot(x, w4[...], preferred_element_type=jnp.float32)
                    + b4[...])


def _mlp(x, W0, b0, W1, b1, W2, b2, W3, b3, W4, b4):
    bb = 512
    n_out = W4.shape[1]
    grid = (_B // bb,)

    def wspec(w):
        return pl.BlockSpec(w.shape, lambda i: (0,) * w.ndim)

    return pl.pallas_call(
        _mlp_body,
        grid=grid,
        in_specs=[pl.BlockSpec((bb, _H), lambda i: (i, 0))]
        + [wspec(w) for w in (W0, b0, W1, b1, W2, b2, W3, b3, W4, b4)],
        out_specs=pl.BlockSpec((bb, n_out), lambda i: (i, 0)),
        out_shape=jax.ShapeDtypeStruct((_B, n_out), jnp.float32),
    )(x, W0, b0, W1, b1, W2, b2, W3, b3, W4, b4)


def kernel(entity, emb, W0, b0, W1, b1, W2, b2, W3, b3, W4, b4):
    emb_lin = _detile(emb.T).reshape(_VROWS, _H)
    pooled = _pool(_remap(entity), emb_lin)
    return _mlp(pooled, W0, b0, W1, b1, W2, b2, W3, b3, W4, b4)


# speedup vs baseline: 1.2716x; 1.0380x over previous
import functools

import jax
import jax.numpy as jnp
from jax import lax
from jax.experimental import pallas as pl
from jax.experimental.pallas import tpu as pltpu
from jax.experimental.pallas import tpu_sc as plsc

_B = 4096
_T = 200
_H = 64
_NW = 32
_RPW = _B // _NW
_NBUF = 4
_LANES = 16
_CCH = _H // _LANES
_TC0, _TC1 = 128, _T - 128


def _pool_body(entity_hbm, table_hbm, out_hbm, eidx, bufs, out_v, sems):
    c = lax.axis_index("c")
    s = lax.axis_index("s")
    wid = s * 2 + c
    base = wid * _RPW

    pltpu.sync_copy(entity_hbm.at[pl.ds(base, _RPW)], eidx)

    inv_t = jnp.float32(1.0 / _T)

    def issue(row, buf, sem):
        r = jnp.minimum(row, _RPW - 1)
        d0 = pltpu.async_copy(
            table_hbm.at[eidx.at[r, pl.ds(0, _TC0)]],
            buf.at[pl.ds(0, _TC0)], sem)
        d1 = pltpu.async_copy(
            table_hbm.at[eidx.at[r, pl.ds(_TC0, _TC1)]],
            buf.at[pl.ds(_TC0, _TC1)], sem)
        return d0, d1

    def accum(row, buf):
        def tstep(t, accs):
            return tuple(
                accs[cc] + buf[t, pl.ds(cc * _LANES, _LANES)]
                for cc in range(_CCH))
        accs = lax.fori_loop(
            0, _T, tstep,
            tuple(jnp.zeros((_LANES,), jnp.float32) for _ in range(_CCH)),
            unroll=2)
        for cc in range(_CCH):
            out_v[row, pl.ds(cc * _LANES, _LANES)] = accs[cc] * inv_t

    def group(i, carry):
        r0 = i * _NBUF
        descs = []
        for k in range(_NBUF):
            descs.append(issue(r0 + k, bufs[k], sems[k]))
        for k in range(_NBUF):
            d0, d1 = descs[k]
            d0.wait()
            d1.wait()
            accum(r0 + k, bufs[k])
        return carry

    lax.fori_loop(0, _RPW // _NBUF, group, 0)
    pltpu.sync_copy(out_v, out_hbm.at[pl.ds(base, _RPW)])


@functools.partial(jax.jit, static_argnums=())
def _pool(entity, emb):
    mesh = plsc.VectorSubcoreMesh(core_axis_name="c", subcore_axis_name="s")
    f = pl.kernel(
        _pool_body,
        out_type=jax.ShapeDtypeStruct((_B, _H), jnp.float32),
        mesh=mesh,
        scratch_types=[
            pltpu.VMEM((_RPW, _T), jnp.int32),
            [pltpu.VMEM((_T, _H), jnp.float32) for _ in range(_NBUF)],
            pltpu.VMEM((_RPW, _H), jnp.float32),
            [pltpu.SemaphoreType.DMA for _ in range(_NBUF)],
        ],
        compiler_params=pltpu.CompilerParams(use_tc_tiling_on_sc=False),
    )
    return f(entity, emb)


_VB = 32768
_V = 1000000
_NBLK = (_V + _VB - 1) // _VB
_VROWS = _NBLK * _VB


def _tr_body(in_ref, out_ref):
    x = in_ref[...]
    xt = jnp.transpose(x, (1, 0))
    out_ref[...] = jnp.concatenate(
        [xt[: _VB // 2], xt[_VB // 2:]], axis=1)


def _detile(embT):
    return pl.pallas_call(
        _tr_body,
        grid=(_NBLK,),
        in_specs=[pl.BlockSpec((_H, _VB), lambda i: (0, i))],
        out_specs=pl.BlockSpec((_VB // 2, 128), lambda i: (i, 0)),
        out_shape=jax.ShapeDtypeStruct((_VROWS // 2, 128), jnp.float32),
        compiler_params=pltpu.CompilerParams(
            dimension_semantics=("parallel",)),
    )(embT)


def _remap(entity):
    r = entity & (_VB - 1)
    return entity + jnp.where(r < _VB // 2, r, r - (_VB - 1))


def _mlp_body(x_ref, w0, b0, w1, b1, w2, b2, w3, b3, w4, b4, out_ref):
    x = x_ref[...]
    x = jnp.maximum(jnp.dot(x, w0[...], preferred_element_type=jnp.float32)
                    + b0[...], 0.0)
    x = jnp.maximum(jnp.dot(x, w1[...], preferred_element_type=jnp.float32)
                    + b1[...], 0.0)
    x = jnp.maximum(jnp.dot(x, w2[...], preferred_element_type=jnp.float32)
                    + b2[...], 0.0)
    x = jnp.maximum(jnp.dot(x, w3[...], preferred_element_type=jnp.float32)
                    + b3[...], 0.0)
    out_ref[...] = (jnp.dot(x, w4[...], preferred_element_type=jnp.float32)
                    + b4[...])


def _mlp(x, W0, b0, W1, b1, W2, b2, W3, b3, W4, b4):
    bb = 512
    n_out = W4.shape[1]
    grid = (_B // bb,)

    def wspec(w):
        return pl.BlockSpec(w.shape, lambda i: (0,) * w.ndim)

    return pl.pallas_call(
        _mlp_body,
        grid=grid,
        in_specs=[pl.BlockSpec((bb, _H), lambda i: (i, 0))]
        + [wspec(w) for w in (W0, b0, W1, b1, W2, b2, W3, b3, W4, b4)],
        out_specs=pl.BlockSpec((bb, n_out), lambda i: (i, 0)),
        out_shape=jax.ShapeDtypeStruct((_B, n_out), jnp.float32),
    )(x, W0, b0, W1, b1, W2, b2, W3, b3, W4, b4)


def kernel(entity, emb, W0, b0, W1, b1, W2, b2, W3, b3, W4, b4):
    emb_lin = _detile(emb.T).reshape(_VROWS, _H)
    pooled = _pool(_remap(entity), emb_lin)
    return _mlp(pooled, W0, b0, W1, b1, W2, b2, W3, b3, W4, b4)
